# speedup vs baseline: 3.0916x; 3.0916x over previous
import functools

import jax
import jax.numpy as jnp
from jax import lax
from jax.experimental import pallas as pl
from jax.experimental.pallas import tpu as pltpu
from jax.experimental.pallas import tpu_sc as plsc

N = 10000
E = 320000
D = 128
H = 256
DO = 128

NW = 32
K = 128
EP = 327680
PW = EP // NW
CH = PW // K
R = 10240
ZR = R // 16

_mesh = plsc.VectorSubcoreMesh(core_axis_name="c", subcore_axis_name="s")


@functools.partial(
    pl.kernel,
    mesh=_mesh,
    out_type=jax.ShapeDtypeStruct((2, R, D), jnp.float32),
    scratch_types=[
        pltpu.VMEM((K,), jnp.int32),
        pltpu.VMEM((K,), jnp.int32),
        pltpu.VMEM((K, D), jnp.float32),
        pltpu.VMEM_SHARED((R, D), jnp.float32),
        pltpu.SemaphoreType.DMA,
    ],
)
def _sc_seg_sum(x_hbm, src_hbm, dst_hbm, zeros_hbm, out_hbm,
                sidx, didx, rows, acc, sem):
    c = lax.axis_index("c")
    s = lax.axis_index("s")
    wid = s * 2 + c
    pltpu.sync_copy(zeros_hbm, acc.at[pl.ds(s * ZR, ZR)])
    plsc.subcore_barrier()

    base = wid * PW

    def body(i, carry):
        off = base + i * K
        pltpu.sync_copy(src_hbm.at[pl.ds(off, K)], sidx)
        pltpu.sync_copy(dst_hbm.at[pl.ds(off, K)], didx)
        pltpu.async_copy(x_hbm.at[sidx], rows, sem).wait()
        pltpu.sync_copy(rows, acc.at[didx], add=True)
        return carry

    lax.fori_loop(0, CH, body, 0)

    plsc.subcore_barrier()
    pltpu.sync_copy(acc.at[pl.ds(s * ZR, ZR)], out_hbm.at[c, pl.ds(s * ZR, ZR)])


def _mlp_body(x_ref, p0_ref, p1_ref, w1_ref, b1_ref, w2_ref, b2_ref, o_ref):
    feat = x_ref[...] + p0_ref[...] + p1_ref[...]
    h = jnp.dot(feat, w1_ref[...], preferred_element_type=jnp.float32)
    h = jnp.maximum(h + b1_ref[...], 0.0)
    o = jnp.dot(h, w2_ref[...], preferred_element_type=jnp.float32)
    o_ref[...] = o + b2_ref[...]


BM = 2000


def _tc_mlp(x, p0, p1, w1t, b1, w2t, b2):
    return pl.pallas_call(
        _mlp_body,
        grid=(N // BM,),
        in_specs=[
            pl.BlockSpec((BM, D), lambda i: (i, 0)),
            pl.BlockSpec((BM, D), lambda i: (i, 0)),
            pl.BlockSpec((BM, D), lambda i: (i, 0)),
            pl.BlockSpec((D, H), lambda i: (0, 0)),
            pl.BlockSpec((1, H), lambda i: (0, 0)),
            pl.BlockSpec((H, DO), lambda i: (0, 0)),
            pl.BlockSpec((1, DO), lambda i: (0, 0)),
        ],
        out_specs=pl.BlockSpec((BM, DO), lambda i: (i, 0)),
        out_shape=jax.ShapeDtypeStruct((N, DO), jnp.float32),
    )(x, p0, p1, w1t, b1, w2t, b2)


def kernel(x, edge_index, W1, b1, W2, b2):
    src = edge_index[0].astype(jnp.int32)
    dst = edge_index[1].astype(jnp.int32)
    pad = EP - E
    src_p = jnp.concatenate([src, jnp.zeros((pad,), jnp.int32)])
    dst_p = jnp.concatenate([dst, jnp.full((pad,), N, jnp.int32)])
    zeros = jnp.zeros((ZR, D), jnp.float32)
    parts = _sc_seg_sum(x, src_p, dst_p, zeros)
    p0 = parts[0, :N]
    p1 = parts[1, :N]
    return _tc_mlp(x, p0, p1, W1.T, b1.reshape(1, H), W2.T, b2.reshape(1, DO))


# speedup vs baseline: 3.7504x; 1.2131x over previous
import functools

import jax
import jax.numpy as jnp
from jax import lax
from jax.experimental import pallas as pl
from jax.experimental.pallas import tpu as pltpu
from jax.experimental.pallas import tpu_sc as plsc

N = 10000
E = 320000
D = 128
H = 256
DO = 128

NW = 32
K = 80
CH = 128
EP = NW * CH * K
PW = CH * K
R = 10240
ZR = R // 16

_mesh = plsc.VectorSubcoreMesh(core_axis_name="c", subcore_axis_name="s")


@functools.partial(
    pl.kernel,
    mesh=_mesh,
    out_type=jax.ShapeDtypeStruct((2, R, D), jnp.float32),
    scratch_types=(
        [pltpu.VMEM((K,), jnp.int32) for _ in range(8)]
        + [pltpu.VMEM((K,), jnp.int32) for _ in range(8)]
        + [pltpu.VMEM((K, D), jnp.float32) for _ in range(4)]
        + [pltpu.VMEM_SHARED((R, D), jnp.float32)]
        + [pltpu.SemaphoreType.DMA for _ in range(16)]
    ),
)
def _sc_seg_sum(x_hbm, src_hbm, dst_hbm, zeros_hbm, out_hbm, *sc):
    sidx = sc[0:8]
    didx = sc[8:16]
    rows = sc[16:20]
    acc = sc[20]
    g = sc[21:25]
    s = sc[25:29]
    isem = sc[29:37]

    cx = lax.axis_index("c")
    sx = lax.axis_index("s")
    wid = sx * 2 + cx
    for z in range(ZR // 128):
        pltpu.sync_copy(zeros_hbm, acc.at[pl.ds(sx * ZR + z * 128, 128)])
    plsc.subcore_barrier()

    for cc in range(4):
        pltpu.async_copy(src_hbm.at[wid, cc], sidx[cc], isem[cc])
        pltpu.async_copy(dst_hbm.at[wid, cc], didx[cc], isem[cc])
    for cc in range(2):
        pltpu.make_async_copy(src_hbm.at[wid, cc], sidx[cc], isem[cc]).wait()
        pltpu.make_async_copy(dst_hbm.at[wid, cc], didx[cc], isem[cc]).wait()
        pltpu.async_copy(x_hbm.at[sidx[cc]], rows[cc], g[cc])

    def body(j, carry):
        for k in range(8):
            c = 8 * j + k
            kr = k % 4
            k2r = (k + 2) % 4
            k2 = (k + 2) % 8
            k4 = (k + 4) % 8
            k6 = (k + 6) % 8

            def drain_cm2():
                pltpu.make_async_copy(
                    rows[k2r], acc.at[didx[k6]], s[k2r]).wait()

            if k >= 2:
                drain_cm2()
            else:
                pl.when(j > 0)(drain_cm2)

            def gather_cp2():
                pltpu.make_async_copy(
                    src_hbm.at[wid, c + 2], sidx[k2], isem[k2]).wait()
                pltpu.make_async_copy(
                    dst_hbm.at[wid, c + 2], didx[k2], isem[k2]).wait()
                pltpu.async_copy(x_hbm.at[sidx[k2]], rows[k2r], g[k2r])

            if k < 6:
                gather_cp2()
            else:
                pl.when(j < CH // 8 - 1)(gather_cp2)

            pltpu.make_async_copy(x_hbm.at[sidx[k]], rows[kr], g[kr]).wait()
            pltpu.async_copy(rows[kr], acc.at[didx[k]], s[kr], add=True)

            def idx_cp4():
                pltpu.async_copy(src_hbm.at[wid, c + 4], sidx[k4], isem[k4])
                pltpu.async_copy(dst_hbm.at[wid, c + 4], didx[k4], isem[k4])

            if k < 4:
                idx_cp4()
            else:
                pl.when(j < CH // 8 - 1)(idx_cp4)
        return carry

    lax.fori_loop(0, CH // 8, body, 0)

    pltpu.make_async_copy(rows[2], acc.at[didx[6]], s[2]).wait()
    pltpu.make_async_copy(rows[3], acc.at[didx[7]], s[3]).wait()

    plsc.subcore_barrier()
    pltpu.sync_copy(acc.at[pl.ds(sx * ZR, ZR)], out_hbm.at[cx, pl.ds(sx * ZR, ZR)])


def _mlp_body(x_ref, p0_ref, p1_ref, w1_ref, b1_ref, w2_ref, b2_ref, o_ref):
    feat = x_ref[...] + p0_ref[...] + p1_ref[...]
    h = jnp.dot(feat, w1_ref[...], preferred_element_type=jnp.float32)
    h = jnp.maximum(h + b1_ref[...], 0.0)
    o = jnp.dot(h, w2_ref[...], preferred_element_type=jnp.float32)
    o_ref[...] = o + b2_ref[...]


BM = 2000


def _tc_mlp(x, p0, p1, w1t, b1, w2t, b2):
    return pl.pallas_call(
        _mlp_body,
        grid=(N // BM,),
        in_specs=[
            pl.BlockSpec((BM, D), lambda i: (i, 0)),
            pl.BlockSpec((BM, D), lambda i: (i, 0)),
            pl.BlockSpec((BM, D), lambda i: (i, 0)),
            pl.BlockSpec((D, H), lambda i: (0, 0)),
            pl.BlockSpec((1, H), lambda i: (0, 0)),
            pl.BlockSpec((H, DO), lambda i: (0, 0)),
            pl.BlockSpec((1, DO), lambda i: (0, 0)),
        ],
        out_specs=pl.BlockSpec((BM, DO), lambda i: (i, 0)),
        out_shape=jax.ShapeDtypeStruct((N, DO), jnp.float32),
    )(x, p0, p1, w1t, b1, w2t, b2)


def kernel(x, edge_index, W1, b1, W2, b2):
    src = edge_index[0].astype(jnp.int32)
    dst = edge_index[1].astype(jnp.int32)
    pad = EP - E
    src_p = jnp.concatenate([src, jnp.zeros((pad,), jnp.int32)])
    dst_p = jnp.concatenate(
        [dst, N + (jnp.arange(pad, dtype=jnp.int32) % (R - N))])
    zeros = jnp.zeros((128, D), jnp.float32)
    parts = _sc_seg_sum(x, src_p.reshape(NW, CH, K), dst_p.reshape(NW, CH, K),
                        zeros)
    p0 = parts[0, :N]
    p1 = parts[1, :N]
    return _tc_mlp(x, p0, p1, W1.T, b1.reshape(1, H), W2.T, b2.reshape(1, DO))
